# baseline (speedup 1.0000x reference)
import functools

import jax
import jax.numpy as jnp
from jax import lax
from jax.experimental import pallas as pl
from jax.experimental.pallas import tpu as pltpu
from jax.experimental.pallas import tpu_sc as plsc

B = 16384
V = 10
D = 784
NC = 2
NS = 16
NW = NC * NS
BPW = B // NW
C = 64
NCHUNK = BPW // C


def _make_sc_gather():
    mesh = plsc.VectorSubcoreMesh(core_axis_name="c", subcore_axis_name="s")

    @functools.partial(
        pl.kernel,
        mesh=mesh,
        compiler_params=pltpu.CompilerParams(use_tc_tiling_on_sc=False),
        out_type=jax.ShapeDtypeStruct((B, 1, 28, 28), jnp.float32),
        scratch_types=[
            pltpu.VMEM((BPW,), jnp.int32),
            pltpu.VMEM((2, C, 1, 28, 28), jnp.float32),
            pltpu.SemaphoreType.DMA,
            pltpu.SemaphoreType.DMA,
            pltpu.SemaphoreType.DMA,
            pltpu.SemaphoreType.DMA,
        ],
    )
    def k(idx_hbm, table_hbm, out_hbm, idx_v, buf_v, gsem0, gsem1, ssem0, ssem1):
        wid = lax.axis_index("s") * NC + lax.axis_index("c")
        base = wid * BPW
        pltpu.sync_copy(idx_hbm.at[pl.ds(base, BPW)], idx_v)

        gsems = (gsem0, gsem1)
        ssems = (ssem0, ssem1)
        gathers = [None, None]
        outs = [None, None]
        gathers[0] = pltpu.async_copy(
            table_hbm.at[idx_v.at[pl.ds(0, C)]], buf_v.at[0], gsems[0])
        for kk in range(NCHUNK):
            cur = kk % 2
            nxt = (kk + 1) % 2
            gathers[cur].wait()
            if kk + 1 < NCHUNK:
                if outs[nxt] is not None:
                    outs[nxt].wait()
                gathers[nxt] = pltpu.async_copy(
                    table_hbm.at[idx_v.at[pl.ds((kk + 1) * C, C)]],
                    buf_v.at[nxt], gsems[nxt])
            outs[cur] = pltpu.async_copy(
                buf_v.at[cur], out_hbm.at[pl.ds(base + kk * C, C)], ssems[cur])
        outs[0].wait()
        outs[1].wait()

    return k


_sc_gather = _make_sc_gather()


def kernel(x, logits):
    return _sc_gather(x.astype(jnp.int32), logits)


# speedup vs baseline: 1.3705x; 1.3705x over previous
import functools

import jax
import jax.numpy as jnp
from jax import lax
from jax.experimental import pallas as pl
from jax.experimental.pallas import tpu as pltpu
from jax.experimental.pallas import tpu_sc as plsc

B = 16384
V = 10
NC = 2
NS = 16
NW = NC * NS
BPW = B // NW
G = 16
NGRP = BPW // G


def _make_sc_gather():
    mesh = plsc.VectorSubcoreMesh(core_axis_name="c", subcore_axis_name="s")

    @functools.partial(
        pl.kernel,
        mesh=mesh,
        out_type=jax.ShapeDtypeStruct((B, 1, 28, 28), jnp.float32),
        scratch_types=[
            pltpu.VMEM((BPW,), jnp.int32),
            pltpu.VMEM((V, 1, 28, 28), jnp.float32),
            pltpu.SemaphoreType.DMA,
            pltpu.SemaphoreType.DMA,
        ],
    )
    def k(idx_hbm, table_hbm, out_hbm, idx_v, table_v, tsem, osem):
        wid = lax.axis_index("s") * NC + lax.axis_index("c")
        base = wid * BPW
        pltpu.sync_copy(idx_hbm.at[pl.ds(base, BPW)], idx_v)
        pltpu.async_copy(table_hbm, table_v, tsem).wait()

        def body(g, _):
            iv = idx_v[pl.ds(g * G, G)]
            copies = []
            for j in range(G):
                s = iv[j]
                copies.append(pltpu.async_copy(
                    table_v.at[pl.ds(s, 1)],
                    out_hbm.at[pl.ds(base + g * G + j, 1)],
                    osem))
            for c in copies:
                c.wait()
            return 0

        lax.fori_loop(0, NGRP, body, 0)

    return k


_sc_gather = _make_sc_gather()


def kernel(x, logits):
    return _sc_gather(x.astype(jnp.int32), logits)


# speedup vs baseline: 1.8196x; 1.3276x over previous
import functools

import jax
import jax.numpy as jnp
from jax import lax
from jax.experimental import pallas as pl
from jax.experimental.pallas import tpu as pltpu
from jax.experimental.pallas import tpu_sc as plsc

B = 16384
V = 10
D = 784
NC = 2
NS = 16
NW = NC * NS
BPW = B // NW
L = 16
NG = BPW // L
H = 28


def _make_sc_gather():
    mesh = plsc.VectorSubcoreMesh(core_axis_name="c", subcore_axis_name="s")

    @functools.partial(
        pl.kernel,
        mesh=mesh,
        compiler_params=pltpu.CompilerParams(
            needs_layout_passes=False, use_tc_tiling_on_sc=False),
        out_type=jax.ShapeDtypeStruct((H, 28, 1, B), jnp.float32),
        scratch_types=[
            pltpu.VMEM((BPW,), jnp.int32),
            pltpu.VMEM((V * D,), jnp.float32),
            pltpu.VMEM((2, 1, 28, 1, BPW), jnp.float32),
            pltpu.SemaphoreType.DMA,
            pltpu.SemaphoreType.DMA,
            pltpu.SemaphoreType.DMA,
        ],
    )
    def k(idx_hbm, table_hbm, out_hbm, idx_v, table_v, buf_v, tsem, osem0, osem1):
        wid = lax.axis_index("s") * NC + lax.axis_index("c")
        base = wid * BPW
        pltpu.sync_copy(idx_hbm.at[pl.ds(base, BPW)], idx_v)
        pltpu.async_copy(table_hbm, table_v, tsem).wait()

        osems = (osem0, osem1)
        outcopies = [None, None]

        for h in range(H):
            cur = h % 2
            if outcopies[cur] is not None:
                outcopies[cur].wait()

            def body(g, _):
                iv = idx_v[pl.ds(g * L, L)]
                rowoff = iv * D + h * 28
                for w in range(28):
                    vals = plsc.load_gather(table_v, [rowoff + w])
                    buf_v[cur, 0, w, 0, pl.ds(g * L, L)] = vals
                return 0

            lax.fori_loop(0, NG, body, 0)
            outcopies[cur] = pltpu.async_copy(
                buf_v.at[cur],
                out_hbm.at[pl.ds(h, 1), :, :, pl.ds(base, BPW)],
                osems[cur])
        outcopies[0].wait()
        outcopies[1].wait()

    return k


_sc_gather = _make_sc_gather()


def kernel(x, logits):
    out_t = _sc_gather(x.astype(jnp.int32), logits.reshape(V * D))
    return jnp.transpose(out_t, (3, 2, 0, 1))


# speedup vs baseline: 3.3062x; 1.8170x over previous
import functools

import jax
import jax.numpy as jnp
from jax import lax
from jax.experimental import pallas as pl
from jax.experimental.pallas import tpu as pltpu
from jax.experimental.pallas import tpu_sc as plsc

B = 16384
V = 10
D = 784
NC = 2
NS = 16
NW = NC * NS
BPW = B // NW
L = 16
NG = BPW // L
H = 28


def _make_sc_gather():
    mesh = plsc.VectorSubcoreMesh(core_axis_name="c", subcore_axis_name="s")

    @functools.partial(
        pl.kernel,
        mesh=mesh,
        compiler_params=pltpu.CompilerParams(
            needs_layout_passes=False, use_tc_tiling_on_sc=False),
        out_type=jax.ShapeDtypeStruct((H, 28, 1, B), jnp.float32),
        scratch_types=[
            pltpu.VMEM((BPW,), jnp.int32),
            pltpu.VMEM((D * L,), jnp.float32),
            pltpu.VMEM((2, 1, 28, 1, BPW), jnp.float32),
            pltpu.SemaphoreType.DMA,
            pltpu.SemaphoreType.DMA,
            pltpu.SemaphoreType.DMA,
        ],
    )
    def k(idx_hbm, table_hbm, out_hbm, idx_v, table_v, buf_v, tsem, osem0, osem1):
        wid = lax.axis_index("s") * NC + lax.axis_index("c")
        base = wid * BPW
        pltpu.sync_copy(idx_hbm.at[pl.ds(base, BPW)], idx_v)
        pltpu.async_copy(table_hbm, table_v, tsem).wait()

        osems = (osem0, osem1)
        outcopies = [None, None]

        for h in range(H):
            cur = h % 2
            if outcopies[cur] is not None:
                outcopies[cur].wait()

            def body(g, _):
                iv = idx_v[pl.ds(g * L, L)]
                for w in range(28):
                    vals = plsc.load_gather(table_v, [iv + (h * 28 + w) * L])
                    buf_v[cur, 0, w, 0, pl.ds(g * L, L)] = vals
                return 0

            lax.fori_loop(0, NG, body, 0)
            outcopies[cur] = pltpu.async_copy(
                buf_v.at[cur],
                out_hbm.at[pl.ds(h, 1), :, :, pl.ds(base, BPW)],
                osems[cur])
        outcopies[0].wait()
        outcopies[1].wait()

    return k


_sc_gather = _make_sc_gather()


def kernel(x, logits):
    table_t = jnp.pad(logits.reshape(V, D).T, ((0, 0), (0, L - V)))
    out_t = _sc_gather(x.astype(jnp.int32), table_t.reshape(D * L))
    return jnp.transpose(out_t, (3, 2, 0, 1))


# speedup vs baseline: 8.9679x; 2.7124x over previous
import functools

import jax
import jax.numpy as jnp
from jax import lax
from jax.experimental import pallas as pl
from jax.experimental.pallas import tpu as pltpu
from jax.experimental.pallas import tpu_sc as plsc

B = 16384
V = 10
D = 784
NC = 2
NS = 16
NW = NC * NS
BPW = B // NW
L = 16
NG = BPW // L
H = 28
SH = 2
NSLAB = H // SH


def _make_sc_gather():
    mesh = plsc.VectorSubcoreMesh(core_axis_name="c", subcore_axis_name="s")

    @functools.partial(
        pl.kernel,
        mesh=mesh,
        compiler_params=pltpu.CompilerParams(
            needs_layout_passes=False, use_tc_tiling_on_sc=False),
        out_type=jax.ShapeDtypeStruct((H, 28, 1, B), jnp.float32),
        scratch_types=[
            pltpu.VMEM((BPW,), jnp.int32),
            pltpu.VMEM((D * L,), jnp.float32),
            pltpu.VMEM((2, SH, 28, 1, BPW), jnp.float32),
            pltpu.SemaphoreType.DMA,
            pltpu.SemaphoreType.DMA,
            pltpu.SemaphoreType.DMA,
        ],
    )
    def k(idx_hbm, table_hbm, out_hbm, idx_v, table_v, buf_v, tsem, osem0, osem1):
        wid = lax.axis_index("s") * NC + lax.axis_index("c")
        base = wid * BPW
        pltpu.sync_copy(idx_hbm.at[pl.ds(base, BPW)], idx_v)
        pltpu.async_copy(table_hbm, table_v, tsem).wait()

        osems = (osem0, osem1)
        outcopies = [None, None]

        for sl in range(NSLAB):
            cur = sl % 2
            if outcopies[cur] is not None:
                outcopies[cur].wait()

            @plsc.parallel_loop(0, NG * SH)
            def body(t):
                g = t // SH
                hh = t % SH
                iv = idx_v[pl.ds(g * L, L)]
                s0 = (sl * SH + hh) * (28 * L)
                for w in range(28):
                    vals = plsc.load_gather(table_v, [iv + (s0 + w * L)])
                    buf_v[cur, hh, w, 0, pl.ds(g * L, L)] = vals
            outcopies[cur] = pltpu.async_copy(
                buf_v.at[cur],
                out_hbm.at[pl.ds(sl * SH, SH), :, :, pl.ds(base, BPW)],
                osems[cur])
        outcopies[0].wait()
        outcopies[1].wait()

    return k


_sc_gather = _make_sc_gather()


def kernel(x, logits):
    table_t = jnp.pad(logits.reshape(V, D).T, ((0, 0), (0, L - V)))
    out_t = _sc_gather(x.astype(jnp.int32), table_t.reshape(D * L))
    return jnp.transpose(out_t, (3, 2, 0, 1))


# speedup vs baseline: 9.9322x; 1.1075x over previous
import functools

import jax
import jax.numpy as jnp
from jax import lax
from jax.experimental import pallas as pl
from jax.experimental.pallas import tpu as pltpu
from jax.experimental.pallas import tpu_sc as plsc

B = 16384
V = 10
D = 784
NC = 2
NS = 16
NW = NC * NS
BPW = B // NW
L = 16
NG = BPW // L
H = 28
SH = 4
NSLAB = H // SH


def _make_sc_gather():
    mesh = plsc.VectorSubcoreMesh(core_axis_name="c", subcore_axis_name="s")

    @functools.partial(
        pl.kernel,
        mesh=mesh,
        compiler_params=pltpu.CompilerParams(
            needs_layout_passes=False, use_tc_tiling_on_sc=False),
        out_type=jax.ShapeDtypeStruct((H, 28, 1, B), jnp.float32),
        scratch_types=[
            pltpu.VMEM((BPW,), jnp.int32),
            pltpu.VMEM((D * L,), jnp.float32),
            pltpu.VMEM((2, SH, 28, 1, BPW), jnp.float32),
            pltpu.SemaphoreType.DMA,
            pltpu.SemaphoreType.DMA,
            pltpu.SemaphoreType.DMA,
            pltpu.SemaphoreType.DMA,
        ],
    )
    def k(idx_hbm, table_hbm, out_hbm, idx_v, table_v, buf_v,
          isem, tsem, osem0, osem1):
        wid = lax.axis_index("s") * NC + lax.axis_index("c")
        base = wid * BPW
        icopy = pltpu.async_copy(idx_hbm.at[pl.ds(base, BPW)], idx_v, isem)
        pltpu.async_copy(table_hbm, table_v, tsem).wait()
        icopy.wait()

        osems = (osem0, osem1)
        outcopies = [None, None]

        for sl in range(NSLAB):
            cur = sl % 2
            if outcopies[cur] is not None:
                outcopies[cur].wait()

            @plsc.parallel_loop(0, NG * SH)
            def body(t):
                g = t // SH
                hh = t % SH
                iv = idx_v[pl.ds(g * L, L)]
                s0 = (sl * SH + hh) * (28 * L)
                for w in range(28):
                    vals = plsc.load_gather(table_v, [iv + (s0 + w * L)])
                    buf_v[cur, hh, w, 0, pl.ds(g * L, L)] = vals
            outcopies[cur] = pltpu.async_copy(
                buf_v.at[cur],
                out_hbm.at[pl.ds(sl * SH, SH), :, :, pl.ds(base, BPW)],
                osems[cur])
        outcopies[0].wait()
        outcopies[1].wait()

    return k


_sc_gather = _make_sc_gather()


def kernel(x, logits):
    table_t = jnp.pad(logits.reshape(V, D).T, ((0, 0), (0, L - V)))
    out_t = _sc_gather(x.astype(jnp.int32), table_t.reshape(D * L))
    return jnp.transpose(out_t, (3, 2, 0, 1))
